# baseline (speedup 1.0000x reference)
import functools

import jax
import jax.numpy as jnp
from jax import lax
from jax.experimental import pallas as pl
from jax.experimental.pallas import tpu as pltpu
from jax.experimental.pallas import tpu_sc as plsc


def _logits_kernel(x_hbm, w1_ref, b1_ref, w2_ref, b2_ref, lo_ref,
                   x_tile, dma_sem, *, nn, bm):
    m = pl.program_id(0)
    n = pl.program_id(1)

    @pl.when(n == 0)
    def _load_x():
        cp = pltpu.make_async_copy(
            x_hbm.at[pl.ds(m * bm, bm), :], x_tile, dma_sem)
        cp.start()
        cp.wait()

    h = jnp.dot(x_tile[...], w1_ref[...], preferred_element_type=jnp.float32)
    h = h + b1_ref[...]
    g = 0.5 * h * (1.0 + jax.lax.erf(h * 0.7071067811865476))
    part = jnp.dot(g, w2_ref[...], preferred_element_type=jnp.float32)

    @pl.when(n == 0)
    def _():
        lo_ref[...] = part + b2_ref[...]

    @pl.when(n != 0)
    def _():
        lo_ref[...] += part


def _routing_sc(logits_hbm, wts_hbm, idx_hbm, upart_hbm,
                lv, wv, iv, uv, *, tpw, nc):
    wid = lax.axis_index("s") * nc + lax.axis_index("c")
    base = wid * tpw
    pltpu.sync_copy(logits_hbm.at[pl.ds(base, tpw), :], lv)

    lane = lax.iota(jnp.int32, 16)
    zeros = jnp.zeros((16,), jnp.float32)
    for cc in range(4):
        uv[cc, :] = zeros

    def body(t, carry):
        raw = []
        vs = []
        ids = []
        for cc in range(4):
            l = lv[t, pl.ds(cc * 16, 16)]
            raw.append(l)
            sk, si = plsc.sort_key_val(l, lane + cc * 16, descending=True)
            vs.append(sk)
            ids.append(si)
        m01v = jnp.where(lane < 8, vs[0], jnp.flip(vs[1]))
        m01i = jnp.where(lane < 8, ids[0], jnp.flip(ids[1]))
        s01v, s01i = plsc.sort_key_val(m01v, m01i, descending=True)
        m23v = jnp.where(lane < 8, vs[2], jnp.flip(vs[3]))
        m23i = jnp.where(lane < 8, ids[2], jnp.flip(ids[3]))
        s23v, s23i = plsc.sort_key_val(m23v, m23i, descending=True)
        mfv = jnp.where(lane < 8, s01v, jnp.flip(s23v))
        mfi = jnp.where(lane < 8, s01i, jnp.flip(s23i))
        fv, fi = plsc.sort_key_val(mfv, mfi, descending=True)
        mx = jnp.max(fv)
        e = jnp.exp(fv - mx)
        denom = jnp.sum(jnp.where(lane < 8, e, zeros))
        wv[t, :] = e / denom
        iv[t, :] = fi
        es = [jnp.exp(r - mx) for r in raw]
        total = jnp.sum(es[0]) + jnp.sum(es[1]) + jnp.sum(es[2]) + jnp.sum(es[3])
        for cc in range(4):
            uv[cc, :] = uv[cc, :] + es[cc] / total
        return carry

    lax.fori_loop(0, tpw, body, 0)

    pltpu.sync_copy(wv, wts_hbm.at[pl.ds(base, tpw), :])
    pltpu.sync_copy(iv, idx_hbm.at[pl.ds(base, tpw), :])
    pltpu.sync_copy(uv, upart_hbm.at[wid])


def kernel(x, W1, b1, W2, b2, training):
    tokens, d_model = x.shape
    hidden = W1.shape[1]
    n_experts = W2.shape[1]
    top_k = 8

    bm = min(2048, tokens)
    bn = min(512, hidden)
    nm = tokens // bm
    nn = hidden // bn

    b1r = b1.reshape(1, hidden)
    b2r = b2.reshape(1, n_experts)

    body = functools.partial(_logits_kernel, nn=nn, bm=bm)

    logits = pl.pallas_call(
        body,
        grid=(nm, nn),
        in_specs=[
            pl.BlockSpec(memory_space=pl.ANY),
            pl.BlockSpec((d_model, bn), lambda m, n: (0, n)),
            pl.BlockSpec((1, bn), lambda m, n: (0, n)),
            pl.BlockSpec((bn, n_experts), lambda m, n: (n, 0)),
            pl.BlockSpec((1, n_experts), lambda m, n: (0, 0)),
        ],
        out_specs=pl.BlockSpec((bm, n_experts), lambda m, n: (m, 0)),
        out_shape=jax.ShapeDtypeStruct((tokens, n_experts), jnp.float32),
        scratch_shapes=[
            pltpu.VMEM((bm, d_model), jnp.float32),
            pltpu.SemaphoreType.DMA,
        ],
    )(x, W1, b1r, W2, b2r)

    try:
        info = plsc.get_sparse_core_info()
        nc, ns = info.num_cores, info.num_subcores
    except Exception:
        nc, ns = 2, 16
    nw = nc * ns
    tpw = tokens // nw

    mesh = plsc.VectorSubcoreMesh(core_axis_name="c", subcore_axis_name="s")
    sc_body = functools.partial(_routing_sc, tpw=tpw, nc=nc)
    wts_pad, idx_pad, upart = pl.kernel(
        sc_body,
        mesh=mesh,
        compiler_params=pltpu.CompilerParams(needs_layout_passes=False),
        out_type=[
            jax.ShapeDtypeStruct((tokens, 16), jnp.float32),
            jax.ShapeDtypeStruct((tokens, 16), jnp.int32),
            jax.ShapeDtypeStruct((nw, 4, 16), jnp.float32),
        ],
        scratch_types=[
            pltpu.VMEM((tpw, n_experts), jnp.float32),
            pltpu.VMEM((tpw, 16), jnp.float32),
            pltpu.VMEM((tpw, 16), jnp.int32),
            pltpu.VMEM((4, 16), jnp.float32),
        ],
    )(logits)

    usage = upart.reshape(nw, n_experts).sum(axis=0) / tokens
    aux = jnp.sum((usage - 1.0 / n_experts) ** 2)
    return wts_pad[:, :top_k], idx_pad[:, :top_k], aux


# speedup vs baseline: 1.0522x; 1.0522x over previous
import functools

import jax
import jax.numpy as jnp
from jax.experimental import pallas as pl
from jax.experimental.pallas import tpu as pltpu


def _gating_kernel(x_hbm, w1_ref, b1_ref, w2_ref, b2_ref,
                   wts_ref, idx_ref, aux_ref,
                   x_tile, logit_acc, usage_acc, dma_sem,
                   *, nm, nn, bm, n_experts, top_k, tokens):
    m = pl.program_id(0)
    n = pl.program_id(1)

    @pl.when(n == 0)
    def _load_x():
        cp = pltpu.make_async_copy(
            x_hbm.at[pl.ds(m * bm, bm), :], x_tile, dma_sem)
        cp.start()
        cp.wait()

    h = jnp.dot(x_tile[...], w1_ref[...], preferred_element_type=jnp.float32)
    h = h + b1_ref[...]
    g = 0.5 * h * (1.0 + jax.lax.erf(h * 0.7071067811865476))
    part = jnp.dot(g, w2_ref[...], preferred_element_type=jnp.float32)

    @pl.when(n == 0)
    def _():
        logit_acc[...] = part

    @pl.when(n != 0)
    def _():
        logit_acc[...] += part

    @pl.when(n == nn - 1)
    def _epilogue():
        lt = (logit_acc[...] + b2_ref[...]).T
        iota_e = jax.lax.broadcasted_iota(jnp.int32, (n_experts, bm), 0)
        cur = lt
        vals = []
        idxs = []
        for _ in range(top_k):
            v = jnp.max(cur, axis=0, keepdims=True)
            i = jnp.min(jnp.where(cur == v, iota_e, n_experts),
                        axis=0, keepdims=True)
            vals.append(v)
            idxs.append(i)
            cur = jnp.where(iota_e == i, -1e30, cur)
        topv = jnp.concatenate(vals, axis=0)
        topi = jnp.concatenate(idxs, axis=0)
        exps = jnp.exp(topv - vals[0])
        wts_t = exps / jnp.sum(exps, axis=0, keepdims=True)
        wts_ref[...] = wts_t
        idx_ref[...] = topi
        p = jnp.exp(lt - vals[0])
        p = p / jnp.sum(p, axis=0, keepdims=True)
        colsum = jnp.sum(p, axis=1, keepdims=True).T

        @pl.when(m == 0)
        def _():
            usage_acc[...] = colsum

        @pl.when(m != 0)
        def _():
            usage_acc[...] += colsum

        @pl.when(m == nm - 1)
        def _final():
            usage = usage_acc[...] / tokens
            diff = usage - (1.0 / n_experts)
            aux_ref[...] = jnp.sum(diff * diff, keepdims=True).reshape(1, 1)


def kernel(x, W1, b1, W2, b2, training):
    tokens, d_model = x.shape
    hidden = W1.shape[1]
    n_experts = W2.shape[1]
    top_k = 8

    bm = min(1024, tokens)
    bn = min(1024, hidden)
    nm = tokens // bm
    nn = hidden // bn

    b1r = b1.reshape(1, hidden)
    b2r = b2.reshape(1, n_experts)

    body = functools.partial(_gating_kernel, nm=nm, nn=nn, bm=bm,
                             n_experts=n_experts, top_k=top_k, tokens=tokens)

    wts, idx, aux = pl.pallas_call(
        body,
        grid=(nm, nn),
        in_specs=[
            pl.BlockSpec(memory_space=pl.ANY),
            pl.BlockSpec((d_model, bn), lambda m, n: (0, n)),
            pl.BlockSpec((1, bn), lambda m, n: (0, n)),
            pl.BlockSpec((bn, n_experts), lambda m, n: (n, 0)),
            pl.BlockSpec((1, n_experts), lambda m, n: (0, 0)),
        ],
        out_specs=[
            pl.BlockSpec((top_k, bm), lambda m, n: (0, m)),
            pl.BlockSpec((top_k, bm), lambda m, n: (0, m)),
            pl.BlockSpec((1, 1), lambda m, n: (0, 0)),
        ],
        out_shape=[
            jax.ShapeDtypeStruct((top_k, tokens), jnp.float32),
            jax.ShapeDtypeStruct((top_k, tokens), jnp.int32),
            jax.ShapeDtypeStruct((1, 1), jnp.float32),
        ],
        scratch_shapes=[
            pltpu.VMEM((bm, d_model), jnp.float32),
            pltpu.VMEM((bm, n_experts), jnp.float32),
            pltpu.VMEM((1, n_experts), jnp.float32),
            pltpu.SemaphoreType.DMA,
        ],
    )(x, W1, b1r, W2, b2r)

    return wts.T, idx.T, aux[0, 0]


# speedup vs baseline: 1.1170x; 1.0616x over previous
import functools

import jax
import jax.numpy as jnp
from jax.experimental import pallas as pl
from jax.experimental.pallas import tpu as pltpu


def _gating_kernel(x_hbm, w1_ref, b1_ref, w2_ref, b2_ref,
                   wts_ref, idx_ref, aux_ref,
                   x_tile, logit_acc, usage_acc, dma_sem,
                   *, nm, nn, bm, n_experts, top_k, tokens):
    m = pl.program_id(0)
    n = pl.program_id(1)

    @pl.when(n == 0)
    def _load_x():
        cp = pltpu.make_async_copy(
            x_hbm.at[pl.ds(m * bm, bm), :], x_tile, dma_sem)
        cp.start()
        cp.wait()

    h = jnp.dot(x_tile[...], w1_ref[...], preferred_element_type=jnp.float32)
    h = h + b1_ref[...]
    g = 0.5 * h * (1.0 + jax.lax.erf(h * 0.7071067811865476))
    part = jnp.dot(g, w2_ref[...], preferred_element_type=jnp.float32)

    @pl.when(n == 0)
    def _():
        logit_acc[...] = part

    @pl.when(n != 0)
    def _():
        logit_acc[...] += part

    @pl.when(n == nn - 1)
    def _epilogue():
        lt = (logit_acc[...] + b2_ref[...]).T
        iota_e = jax.lax.broadcasted_iota(jnp.int32, (n_experts, bm), 0)
        cur = lt
        vals = []
        idxs = []
        for _ in range(top_k):
            v = jnp.max(cur, axis=0, keepdims=True)
            i = jnp.min(jnp.where(cur == v, iota_e, n_experts),
                        axis=0, keepdims=True)
            vals.append(v)
            idxs.append(i)
            cur = jnp.where(iota_e == i, -1e30, cur)
        topv = jnp.concatenate(vals, axis=0)
        topi = jnp.concatenate(idxs, axis=0)
        exps = jnp.exp(topv - vals[0])
        wts_t = exps / jnp.sum(exps, axis=0, keepdims=True)
        wts_ref[...] = wts_t
        idx_ref[...] = topi
        p = jnp.exp(lt - vals[0])
        p = p / jnp.sum(p, axis=0, keepdims=True)
        colsum = jnp.sum(p, axis=1, keepdims=True).T

        @pl.when(m == 0)
        def _():
            usage_acc[...] = colsum

        @pl.when(m != 0)
        def _():
            usage_acc[...] += colsum

        @pl.when(m == nm - 1)
        def _final():
            usage = usage_acc[...] / tokens
            diff = usage - (1.0 / n_experts)
            aux_ref[...] = jnp.sum(diff * diff, keepdims=True).reshape(1, 1)


def kernel(x, W1, b1, W2, b2, training):
    tokens, d_model = x.shape
    hidden = W1.shape[1]
    n_experts = W2.shape[1]
    top_k = 8

    bm = min(2048, tokens)
    bn = min(256, hidden)
    nm = tokens // bm
    nn = hidden // bn

    b1r = b1.reshape(1, hidden)
    b2r = b2.reshape(1, n_experts)

    body = functools.partial(_gating_kernel, nm=nm, nn=nn, bm=bm,
                             n_experts=n_experts, top_k=top_k, tokens=tokens)

    wts, idx, aux = pl.pallas_call(
        body,
        grid=(nm, nn),
        in_specs=[
            pl.BlockSpec(memory_space=pl.ANY),
            pl.BlockSpec((d_model, bn), lambda m, n: (0, n)),
            pl.BlockSpec((1, bn), lambda m, n: (0, n)),
            pl.BlockSpec((bn, n_experts), lambda m, n: (n, 0)),
            pl.BlockSpec((1, n_experts), lambda m, n: (0, 0)),
        ],
        out_specs=[
            pl.BlockSpec((top_k, bm), lambda m, n: (0, m)),
            pl.BlockSpec((top_k, bm), lambda m, n: (0, m)),
            pl.BlockSpec((1, 1), lambda m, n: (0, 0)),
        ],
        out_shape=[
            jax.ShapeDtypeStruct((top_k, tokens), jnp.float32),
            jax.ShapeDtypeStruct((top_k, tokens), jnp.int32),
            jax.ShapeDtypeStruct((1, 1), jnp.float32),
        ],
        scratch_shapes=[
            pltpu.VMEM((bm, d_model), jnp.float32),
            pltpu.VMEM((bm, n_experts), jnp.float32),
            pltpu.VMEM((1, n_experts), jnp.float32),
            pltpu.SemaphoreType.DMA,
        ],
    )(x, W1, b1r, W2, b2r)

    return wts.T, idx.T, aux[0, 0]
